# baseline (speedup 1.0000x reference)
import jax
import jax.numpy as jnp
from jax import lax
from jax.experimental import pallas as pl
from jax.experimental.pallas import tpu as pltpu
from jax.experimental.pallas import tpu_sc as plsc

B = 1024
D = 768
L = 16
NW = 32
RPW = B // NW
HALF = RPW // 2
N_SCORES = 100000
MM_CHUNK = 6256
MM_LAST = N_SCORES - MM_CHUNK
MM_A = 3136
MM_B = MM_CHUNK - MM_A


def _xlane(v, buf, op):
    i = lax.iota(jnp.int32, L)
    for m in (1, 2, 4, 8):
        buf[...] = v
        v = op(v, plsc.load_gather(buf, [i ^ m]))
    return v


def _vsqrt(x):
    i = plsc.bitcast(x, jnp.int32)
    y = plsc.bitcast(jnp.int32(0x5F3759DF) - lax.shift_right_logical(i, 1),
                     jnp.float32)
    for _ in range(3):
        y = y * (1.5 - 0.5 * x * y * y)
    return x * y


def _body(hidx_hbm, ridx_hbm, tidx_hbm, text_hbm, image_hbm, rel_hbm,
          scores_hbm, out_hbm,
          hidx_v, ridx_v, tidx_v, sh_v, st_v,
          th0_v, ih0_v, tt0_v, it0_v, r0_v,
          th1_v, ih1_v, tt1_v, it1_v, r1_v,
          mm_v, mm2_v, part_v, allp_v, shared_mm, xl_v, out_v,
          sem_e0, sem_e1, sem_sc, sem_st):
    emb_bufs = ((th0_v, ih0_v, tt0_v, it0_v, r0_v),
                (th1_v, ih1_v, tt1_v, it1_v, r1_v))
    c = lax.axis_index("c")
    s = lax.axis_index("s")
    wid = s * 2 + c
    base = wid * RPW
    row_i = lax.iota(jnp.int32, L)

    off = jnp.minimum(s * MM_CHUNK, MM_LAST)
    cp_hi = pltpu.async_copy(hidx_hbm.at[pl.ds(base, RPW)], hidx_v, sem_st)
    cp_ri = pltpu.async_copy(ridx_hbm.at[pl.ds(base, RPW)], ridx_v, sem_st)
    cp_ti = pltpu.async_copy(tidx_hbm.at[pl.ds(base, RPW)], tidx_v, sem_st)
    cp_m1 = pltpu.async_copy(scores_hbm.at[pl.ds(off, MM_A)], mm_v, sem_st)
    cp_m2 = pltpu.async_copy(scores_hbm.at[pl.ds(off + MM_A, MM_B)],
                             mm2_v, sem_st)
    cp_hi.wait()
    cp_ri.wait()
    cp_ti.wait()

    def fire_half(half, sem):
        hs = hidx_v.at[pl.ds(half * HALF, HALF)]
        rs = ridx_v.at[pl.ds(half * HALF, HALF)]
        ts = tidx_v.at[pl.ds(half * HALF, HALF)]
        th_v, ih_v, tt_v, it_v, r_v = emb_bufs[half]
        return [
            pltpu.async_copy(text_hbm.at[hs], th_v, sem),
            pltpu.async_copy(image_hbm.at[hs], ih_v, sem),
            pltpu.async_copy(text_hbm.at[ts], tt_v, sem),
            pltpu.async_copy(image_hbm.at[ts], it_v, sem),
            pltpu.async_copy(rel_hbm.at[rs], r_v, sem),
        ]

    cp_sh = pltpu.async_copy(scores_hbm.at[hidx_v], sh_v, sem_sc)
    cp_st = pltpu.async_copy(scores_hbm.at[tidx_v], st_v, sem_sc)
    cps0 = fire_half(0, sem_e0)
    cps1 = fire_half(1, sem_e1)

    def mm_pass(buf, n, carry):
        def step(k, ca):
            vmn, vmx = ca
            v = buf[pl.ds(k * L, L)]
            return jnp.minimum(vmn, v), jnp.maximum(vmx, v)
        return lax.fori_loop(0, n, step, carry, unroll=1)

    cp_m1.wait()
    cp_m2.wait()
    v0 = mm_v[pl.ds(0, L)]
    vmn, vmx = mm_pass(mm_v, MM_A // L, (v0, v0))
    vmn, vmx = mm_pass(mm2_v, MM_B // L, (vmn, vmx))
    part_v[pl.ds(0, L)] = vmn
    part_v[pl.ds(L, L)] = vmx
    pltpu.sync_copy(part_v, shared_mm.at[pl.ds(s * 2 * L, 2 * L)])
    plsc.subcore_barrier()
    pltpu.sync_copy(shared_mm, allp_v)
    for i in range(16):
        vmn = jnp.minimum(vmn, allp_v[pl.ds(i * 2 * L, L)])
        vmx = jnp.maximum(vmx, allp_v[pl.ds(i * 2 * L + L, L)])
    mn = _xlane(vmn, xl_v, jnp.minimum)
    mx = _xlane(vmx, xl_v, jnp.maximum)
    inv_den = 1.0 / (mx - mn)

    cp_sh.wait()
    cp_st.wait()

    for half in range(2):
        th_v, ih_v, tt_v, it_v, r_v = emb_bufs[half]
        for cp in (cps0 if half == 0 else cps1):
            cp.wait()
        hb = half * HALF
        hr_vec = (sh_v[pl.ds(hb, HALF)] - mn) * inv_den
        tr_vec = (st_v[pl.ds(hb, HALF)] - mn) * inv_den
        ah_vec = 1.0 / (1.0 + hr_vec)
        bh_vec = hr_vec * ah_vec
        at_vec = 1.0 / (1.0 + tr_vec)
        bt_vec = tr_vec * at_vec

        @plsc.parallel_loop(0, D, unroll=1,
                            carry=jnp.zeros((L,), jnp.float32))
        def acc(p, a):
            pvec = p + row_i
            pvec = jnp.where(pvec >= D, pvec - D, pvec)
            th = plsc.load_gather(th_v, [row_i, pvec])
            ih = plsc.load_gather(ih_v, [row_i, pvec])
            tt = plsc.load_gather(tt_v, [row_i, pvec])
            it = plsc.load_gather(it_v, [row_i, pvec])
            rl = plsc.load_gather(r_v, [row_i, pvec])
            d = (ah_vec * th + bh_vec * ih + rl
                 - at_vec * tt - bt_vec * it)
            return a + d * d
        out_v[pl.ds(hb, HALF)] = _vsqrt(acc)

    pltpu.sync_copy(out_v, out_hbm.at[pl.ds(base, RPW)])


@jax.jit
def _score(h_idx, r_idx, t_idx, text, image, rel, scores):
    mesh = plsc.VectorSubcoreMesh(core_axis_name="c", subcore_axis_name="s")
    f = pl.kernel(
        _body,
        out_type=jax.ShapeDtypeStruct((B,), jnp.float32),
        mesh=mesh,
        compiler_params=pltpu.CompilerParams(needs_layout_passes=False),
        scratch_types=[
            pltpu.VMEM((RPW,), jnp.int32),
            pltpu.VMEM((RPW,), jnp.int32),
            pltpu.VMEM((RPW,), jnp.int32),
            pltpu.VMEM((RPW,), jnp.float32),
            pltpu.VMEM((RPW,), jnp.float32),
            pltpu.VMEM((HALF, D), jnp.float32),
            pltpu.VMEM((HALF, D), jnp.float32),
            pltpu.VMEM((HALF, D), jnp.float32),
            pltpu.VMEM((HALF, D), jnp.float32),
            pltpu.VMEM((HALF, D), jnp.float32),
            pltpu.VMEM((HALF, D), jnp.float32),
            pltpu.VMEM((HALF, D), jnp.float32),
            pltpu.VMEM((HALF, D), jnp.float32),
            pltpu.VMEM((HALF, D), jnp.float32),
            pltpu.VMEM((HALF, D), jnp.float32),
            pltpu.VMEM((MM_A,), jnp.float32),
            pltpu.VMEM((MM_B,), jnp.float32),
            pltpu.VMEM((2 * L,), jnp.float32),
            pltpu.VMEM((16 * 2 * L,), jnp.float32),
            pltpu.VMEM_SHARED((16 * 2 * L,), jnp.float32),
            pltpu.VMEM((L,), jnp.float32),
            pltpu.VMEM((RPW,), jnp.float32),
            pltpu.SemaphoreType.DMA,
            pltpu.SemaphoreType.DMA,
            pltpu.SemaphoreType.DMA,
            pltpu.SemaphoreType.DMA,
        ],
    )
    return f(h_idx, r_idx, t_idx, text, image, rel, scores)


def kernel(hrt_neighbor, text_embedding, image_embedding, relation_embedding,
           entity_scores):
    pos = hrt_neighbor[:, 0, 0, :].astype(jnp.int32)
    return _score(pos[:, 0], pos[:, 1], pos[:, 2], text_embedding,
                  image_embedding, relation_embedding, entity_scores)


# speedup vs baseline: 1.0112x; 1.0112x over previous
import jax
import jax.numpy as jnp
from jax import lax
from jax.experimental import pallas as pl
from jax.experimental.pallas import tpu as pltpu
from jax.experimental.pallas import tpu_sc as plsc

B = 1024
D = 768
L = 16
NW = 32
RPW = B // NW
HALF = RPW // 2
N_SCORES = 100000
MM_CHUNK = 6256
MM_LAST = N_SCORES - MM_CHUNK
MM_A = 3136
MM_B = MM_CHUNK - MM_A


def _xlane(v, buf, op):
    i = lax.iota(jnp.int32, L)
    for m in (1, 2, 4, 8):
        buf[...] = v
        v = op(v, plsc.load_gather(buf, [i ^ m]))
    return v


def _vsqrt(x):
    i = plsc.bitcast(x, jnp.int32)
    y = plsc.bitcast(jnp.int32(0x5F3759DF) - lax.shift_right_logical(i, 1),
                     jnp.float32)
    for _ in range(3):
        y = y * (1.5 - 0.5 * x * y * y)
    return x * y


def _body(hidx_hbm, ridx_hbm, tidx_hbm, text_hbm, image_hbm, rel_hbm,
          scores_hbm, out_hbm,
          hidx_v, ridx_v, tidx_v, sh_v, st_v,
          th0_v, ih0_v, tt0_v, it0_v, r0_v,
          th1_v, ih1_v, tt1_v, it1_v, r1_v,
          mm_v, mm2_v, part_v, allp_v, shared_mm, xl_v, out_v,
          sem_e0, sem_e1, sem_sc, sem_st):
    emb_bufs = ((th0_v, ih0_v, tt0_v, it0_v, r0_v),
                (th1_v, ih1_v, tt1_v, it1_v, r1_v))
    c = lax.axis_index("c")
    s = lax.axis_index("s")
    wid = s * 2 + c
    base = wid * RPW
    row_i = lax.iota(jnp.int32, L)

    off = jnp.minimum(s * MM_CHUNK, MM_LAST)
    cp_hi = pltpu.async_copy(hidx_hbm.at[pl.ds(base, RPW)], hidx_v, sem_st)
    cp_ri = pltpu.async_copy(ridx_hbm.at[pl.ds(base, RPW)], ridx_v, sem_st)
    cp_ti = pltpu.async_copy(tidx_hbm.at[pl.ds(base, RPW)], tidx_v, sem_st)
    cp_m1 = pltpu.async_copy(scores_hbm.at[pl.ds(off, MM_A)], mm_v, sem_st)
    cp_m2 = pltpu.async_copy(scores_hbm.at[pl.ds(off + MM_A, MM_B)],
                             mm2_v, sem_st)
    cp_hi.wait()
    cp_ri.wait()
    cp_ti.wait()

    def fire_half(half, sem):
        hs = hidx_v.at[pl.ds(half * HALF, HALF)]
        rs = ridx_v.at[pl.ds(half * HALF, HALF)]
        ts = tidx_v.at[pl.ds(half * HALF, HALF)]
        th_v, ih_v, tt_v, it_v, r_v = emb_bufs[half]
        return [
            pltpu.async_copy(text_hbm.at[hs], th_v, sem),
            pltpu.async_copy(image_hbm.at[hs], ih_v, sem),
            pltpu.async_copy(text_hbm.at[ts], tt_v, sem),
            pltpu.async_copy(image_hbm.at[ts], it_v, sem),
            pltpu.async_copy(rel_hbm.at[rs], r_v, sem),
        ]

    cp_sh = pltpu.async_copy(scores_hbm.at[hidx_v], sh_v, sem_sc)
    cp_st = pltpu.async_copy(scores_hbm.at[tidx_v], st_v, sem_sc)
    cps0 = fire_half(0, sem_e0)
    cps1 = fire_half(1, sem_e1)

    def mm_pass(buf, n, carry):
        def step(k, ca):
            vmn, vmx = ca
            v = buf[pl.ds(k * L, L)]
            return jnp.minimum(vmn, v), jnp.maximum(vmx, v)
        return lax.fori_loop(0, n, step, carry, unroll=2)

    cp_m1.wait()
    cp_m2.wait()
    v0 = mm_v[pl.ds(0, L)]
    vmn, vmx = mm_pass(mm_v, MM_A // L, (v0, v0))
    vmn, vmx = mm_pass(mm2_v, MM_B // L, (vmn, vmx))
    part_v[pl.ds(0, L)] = vmn
    part_v[pl.ds(L, L)] = vmx
    pltpu.sync_copy(part_v, shared_mm.at[pl.ds(s * 2 * L, 2 * L)])
    plsc.subcore_barrier()
    pltpu.sync_copy(shared_mm, allp_v)
    for i in range(16):
        vmn = jnp.minimum(vmn, allp_v[pl.ds(i * 2 * L, L)])
        vmx = jnp.maximum(vmx, allp_v[pl.ds(i * 2 * L + L, L)])
    mn = _xlane(vmn, xl_v, jnp.minimum)
    mx = _xlane(vmx, xl_v, jnp.maximum)
    inv_den = 1.0 / (mx - mn)

    cp_sh.wait()
    cp_st.wait()

    for half in range(2):
        th_v, ih_v, tt_v, it_v, r_v = emb_bufs[half]
        for cp in (cps0 if half == 0 else cps1):
            cp.wait()
        hb = half * HALF
        hr_vec = (sh_v[pl.ds(hb, HALF)] - mn) * inv_den
        tr_vec = (st_v[pl.ds(hb, HALF)] - mn) * inv_den
        ah_vec = 1.0 / (1.0 + hr_vec)
        bh_vec = hr_vec * ah_vec
        at_vec = 1.0 / (1.0 + tr_vec)
        bt_vec = tr_vec * at_vec

        @plsc.parallel_loop(0, D, unroll=2,
                            carry=jnp.zeros((L,), jnp.float32))
        def acc(p, a):
            pvec = p + row_i
            pvec = jnp.where(pvec >= D, pvec - D, pvec)
            th = plsc.load_gather(th_v, [row_i, pvec])
            ih = plsc.load_gather(ih_v, [row_i, pvec])
            tt = plsc.load_gather(tt_v, [row_i, pvec])
            it = plsc.load_gather(it_v, [row_i, pvec])
            rl = plsc.load_gather(r_v, [row_i, pvec])
            d = (ah_vec * th + bh_vec * ih + rl
                 - at_vec * tt - bt_vec * it)
            return a + d * d
        out_v[pl.ds(hb, HALF)] = _vsqrt(acc)

    pltpu.sync_copy(out_v, out_hbm.at[pl.ds(base, RPW)])


@jax.jit
def _score(h_idx, r_idx, t_idx, text, image, rel, scores):
    mesh = plsc.VectorSubcoreMesh(core_axis_name="c", subcore_axis_name="s")
    f = pl.kernel(
        _body,
        out_type=jax.ShapeDtypeStruct((B,), jnp.float32),
        mesh=mesh,
        compiler_params=pltpu.CompilerParams(needs_layout_passes=False),
        scratch_types=[
            pltpu.VMEM((RPW,), jnp.int32),
            pltpu.VMEM((RPW,), jnp.int32),
            pltpu.VMEM((RPW,), jnp.int32),
            pltpu.VMEM((RPW,), jnp.float32),
            pltpu.VMEM((RPW,), jnp.float32),
            pltpu.VMEM((HALF, D), jnp.float32),
            pltpu.VMEM((HALF, D), jnp.float32),
            pltpu.VMEM((HALF, D), jnp.float32),
            pltpu.VMEM((HALF, D), jnp.float32),
            pltpu.VMEM((HALF, D), jnp.float32),
            pltpu.VMEM((HALF, D), jnp.float32),
            pltpu.VMEM((HALF, D), jnp.float32),
            pltpu.VMEM((HALF, D), jnp.float32),
            pltpu.VMEM((HALF, D), jnp.float32),
            pltpu.VMEM((HALF, D), jnp.float32),
            pltpu.VMEM((MM_A,), jnp.float32),
            pltpu.VMEM((MM_B,), jnp.float32),
            pltpu.VMEM((2 * L,), jnp.float32),
            pltpu.VMEM((16 * 2 * L,), jnp.float32),
            pltpu.VMEM_SHARED((16 * 2 * L,), jnp.float32),
            pltpu.VMEM((L,), jnp.float32),
            pltpu.VMEM((RPW,), jnp.float32),
            pltpu.SemaphoreType.DMA,
            pltpu.SemaphoreType.DMA,
            pltpu.SemaphoreType.DMA,
            pltpu.SemaphoreType.DMA,
        ],
    )
    return f(h_idx, r_idx, t_idx, text, image, rel, scores)


def kernel(hrt_neighbor, text_embedding, image_embedding, relation_embedding,
           entity_scores):
    pos = hrt_neighbor[:, 0, 0, :].astype(jnp.int32)
    return _score(pos[:, 0], pos[:, 1], pos[:, 2], text_embedding,
                  image_embedding, relation_embedding, entity_scores)
